# baseline (speedup 1.0000x reference)
import functools

import jax
import jax.numpy as jnp
from jax import lax
from jax.experimental import pallas as pl
from jax.experimental.pallas import tpu as pltpu
from jax.experimental.pallas import tpu_sc as plsc

NODE_DIM = 128
EDGE_DIM = 16
N_NODES = 10000
N_EDGES = 320000

NC = 2
NS = 16
NW = NC * NS
NSPLIT = 2
E_CHU = N_EDGES // NSPLIT
PER_W = E_CHU // NW
CH = 40
NCHUNK = PER_W // CH
ROWS_PER_SUB = 624
TAIL_ROWS = N_NODES - NS * ROWS_PER_SUB
TAIL_OFF = NS * ROWS_PER_SUB

BE = 2000
BN = 1000

def _mesh():
    return plsc.VectorSubcoreMesh(
        core_axis_name="c", subcore_axis_name="s", num_cores=NC, num_subcores=NS)


def _softplus(v):
    return jnp.maximum(v, 0.0) + jnp.log(1.0 + jnp.exp(-jnp.abs(v)))


def _gather_body(x_hbm, row_hbm, col_hbm, xr_hbm, xc_hbm,
                 idx_r, idx_c,
                 br0, bc0, br1, bc1, sr0, sc0, sr1, sc1):
    c = lax.axis_index("c")
    s = lax.axis_index("s")
    wid = s * NC + c
    base = wid * PER_W
    pltpu.sync_copy(row_hbm.at[wid], idx_r)
    pltpu.sync_copy(col_hbm.at[wid], idx_c)

    bufs = ((br0, bc0, sr0, sc0), (br1, bc1, sr1, sc1))

    def fire(j, k):
        br, bc, sr, sc = bufs[k]
        pltpu.async_copy(x_hbm.at[idx_r.at[j]], br, sr)
        pltpu.async_copy(x_hbm.at[idx_c.at[j]], bc, sc)

    def drain_write(j, k):
        br, bc, sr, sc = bufs[k]
        pltpu.make_async_copy(x_hbm.at[idx_r.at[j]], br, sr).wait()
        pltpu.make_async_copy(x_hbm.at[idx_c.at[j]], bc, sc).wait()
        off = base + j * CH
        pltpu.sync_copy(br, xr_hbm.at[pl.ds(off, CH)])
        pltpu.sync_copy(bc, xc_hbm.at[pl.ds(off, CH)])

    fire(0, 0)

    def body(t, carry):
        j0 = 2 * t
        fire(j0 + 1, 1)
        drain_write(j0, 0)
        fire(j0 + 2, 0)
        drain_write(j0 + 1, 1)
        return carry

    lax.fori_loop(0, (NCHUNK - 1) // 2, body, 0)
    drain_write(NCHUNK - 1, 0)


@jax.jit
def _gather(x, row3, col3):
    f = pl.kernel(
        _gather_body,
        out_type=(
            jax.ShapeDtypeStruct((E_CHU, NODE_DIM), jnp.float32),
            jax.ShapeDtypeStruct((E_CHU, NODE_DIM), jnp.float32),
        ),
        mesh=_mesh(),
        scratch_types=[
            pltpu.VMEM((NCHUNK, CH), jnp.int32),
            pltpu.VMEM((NCHUNK, CH), jnp.int32),
            pltpu.VMEM((CH, NODE_DIM), jnp.float32),
            pltpu.VMEM((CH, NODE_DIM), jnp.float32),
            pltpu.VMEM((CH, NODE_DIM), jnp.float32),
            pltpu.VMEM((CH, NODE_DIM), jnp.float32),
            pltpu.SemaphoreType.DMA,
            pltpu.SemaphoreType.DMA,
            pltpu.SemaphoreType.DMA,
            pltpu.SemaphoreType.DMA,
        ],
    )
    return f(x, row3, col3)


def _scatter_body(emb_hbm, col_hbm, zeros_hbm, out_hbm,
                  idx_v, b0, b1, shared, s0, s1):
    c = lax.axis_index("c")
    s = lax.axis_index("s")
    wid = s * NC + c
    base = wid * PER_W
    r0 = s * ROWS_PER_SUB
    pltpu.sync_copy(zeros_hbm.at[pl.ds(r0, ROWS_PER_SUB)],
                    shared.at[pl.ds(r0, ROWS_PER_SUB)])

    @pl.when(s == NS - 1)
    def _():
        pltpu.sync_copy(zeros_hbm.at[pl.ds(TAIL_OFF, TAIL_ROWS)],
                        shared.at[pl.ds(TAIL_OFF, TAIL_ROWS)])

    pltpu.sync_copy(col_hbm.at[wid], idx_v)
    plsc.subcore_barrier()

    bufs = ((b0, s0), (b1, s1))

    def fire(j, k):
        b, sm = bufs[k]
        pltpu.async_copy(emb_hbm.at[pl.ds(base + j * CH, CH)], b, sm)

    def drain_add(j, k):
        b, sm = bufs[k]
        pltpu.make_async_copy(emb_hbm.at[pl.ds(base + j * CH, CH)], b,
                              sm).wait()
        pltpu.sync_copy(b, shared.at[idx_v.at[j]], add=True)

    fire(0, 0)

    def body(t, carry):
        j0 = 2 * t
        fire(j0 + 1, 1)
        drain_add(j0, 0)
        fire(j0 + 2, 0)
        drain_add(j0 + 1, 1)
        return carry

    lax.fori_loop(0, (NCHUNK - 1) // 2, body, 0)
    drain_add(NCHUNK - 1, 0)
    plsc.subcore_barrier()
    pltpu.sync_copy(shared.at[pl.ds(r0, ROWS_PER_SUB)],
                    out_hbm.at[c, pl.ds(r0, ROWS_PER_SUB)])

    @pl.when(s == NS - 1)
    def _():
        pltpu.sync_copy(shared.at[pl.ds(TAIL_OFF, TAIL_ROWS)],
                        out_hbm.at[c, pl.ds(TAIL_OFF, TAIL_ROWS)])


@jax.jit
def _scatter(emb, col3, zeros):
    f = pl.kernel(
        _scatter_body,
        out_type=jax.ShapeDtypeStruct((NC, N_NODES, NODE_DIM), jnp.float32),
        mesh=_mesh(),
        scratch_types=[
            pltpu.VMEM((NCHUNK, CH), jnp.int32),
            pltpu.VMEM((CH, NODE_DIM), jnp.float32),
            pltpu.VMEM((CH, NODE_DIM), jnp.float32),
            pltpu.VMEM_SHARED((N_NODES, NODE_DIM), jnp.float32),
            pltpu.SemaphoreType.DMA,
            pltpu.SemaphoreType.DMA,
        ],
    )
    return f(emb, col3, zeros)


def _edge_mlp_body(xr, xc, ea, w1a, w1b, w1c, b1, w2, b2, out):
    bf = jnp.bfloat16
    acc = jnp.dot(xr[...].astype(bf), w1a[...],
                  preferred_element_type=jnp.float32)
    acc += jnp.dot(xc[...].astype(bf), w1b[...],
                   preferred_element_type=jnp.float32)
    acc += jnp.dot(ea[...], w1c[...], preferred_element_type=jnp.float32)
    acc += b1[...]
    h = _softplus(acc)
    o = jnp.dot(h.astype(jnp.bfloat16), w2[...],
                preferred_element_type=jnp.float32) + b2[...]
    out[...] = _softplus(o)


@jax.jit
def _edge_mlp(xr, xc, ea, w1a, w1b, w1c, b1, w2, b2):
    nblk = E_CHU // BE
    full = lambda shape: pl.BlockSpec(shape, lambda i: (0, 0))
    return pl.pallas_call(
        _edge_mlp_body,
        grid=(nblk,),
        in_specs=[
            pl.BlockSpec((BE, NODE_DIM), lambda i: (i, 0)),
            pl.BlockSpec((BE, NODE_DIM), lambda i: (i, 0)),
            pl.BlockSpec((BE, EDGE_DIM), lambda i: (i, 0)),
            full((NODE_DIM, 2 * NODE_DIM)),
            full((NODE_DIM, 2 * NODE_DIM)),
            full((EDGE_DIM, 2 * NODE_DIM)),
            full((1, 2 * NODE_DIM)),
            full((2 * NODE_DIM, NODE_DIM)),
            full((1, NODE_DIM)),
        ],
        out_specs=pl.BlockSpec((BE, NODE_DIM), lambda i: (i, 0)),
        out_shape=jax.ShapeDtypeStruct((E_CHU, NODE_DIM), jnp.float32),
        compiler_params=pltpu.CompilerParams(
            dimension_semantics=("parallel",)),
    )(xr, xc, ea, w1a, w1b, w1c, b1, w2, b2)


def _node_body(x, a0, a1, a2, a3, w3a, w3b, b3, out):
    ag = (a0[...] + a1[...]) + (a2[...] + a3[...])
    o = jnp.dot(x[...], w3a[...], preferred_element_type=jnp.float32)
    o += jnp.dot(ag, w3b[...], preferred_element_type=jnp.float32)
    o += b3[...]
    out[...] = _softplus(o) + x[...]


@jax.jit
def _node(x, a0, a1, a2, a3, w3a, w3b, b3):
    nblk = N_NODES // BN
    full = lambda shape: pl.BlockSpec(shape, lambda i: (0, 0))
    blk = pl.BlockSpec((BN, NODE_DIM), lambda i: (i, 0))
    return pl.pallas_call(
        _node_body,
        grid=(nblk,),
        in_specs=[
            blk, blk, blk, blk, blk,
            full((NODE_DIM, NODE_DIM)),
            full((NODE_DIM, NODE_DIM)),
            full((1, NODE_DIM)),
        ],
        out_specs=pl.BlockSpec((BN, NODE_DIM), lambda i: (i, 0)),
        out_shape=jax.ShapeDtypeStruct((N_NODES, NODE_DIM), jnp.float32),
        compiler_params=pltpu.CompilerParams(
            dimension_semantics=("parallel",)),
    )(x, a0, a1, a2, a3, w3a, w3b, b3)


def kernel(x, edge_index, edge_attr, W1, b1, W2, b2, W3, b3):
    row4 = edge_index[0].astype(jnp.int32).reshape(NSPLIT, NW, NCHUNK, CH)
    col4 = edge_index[1].astype(jnp.int32).reshape(NSPLIT, NW, NCHUNK, CH)
    bf = jnp.bfloat16
    w1a = W1[:NODE_DIM].astype(bf)
    w1b = W1[NODE_DIM:2 * NODE_DIM].astype(bf)
    w1c = W1[2 * NODE_DIM:].astype(bf)
    b1r = b1.reshape(1, -1)
    w2 = W2.astype(bf)
    b2r = b2.reshape(1, -1)
    ea = edge_attr.astype(bf)
    zeros = jnp.zeros((N_NODES, NODE_DIM), jnp.float32)

    parts = []
    for t in range(NSPLIT):
        xr, xc = _gather(x, row4[t], col4[t])
        emb = _edge_mlp(xr, xc, ea[t * E_CHU:(t + 1) * E_CHU],
                        w1a, w1b, w1c, b1r, w2, b2r)
        parts.append(_scatter(emb, col4[t], zeros))
    return _node(x, parts[0][0], parts[0][1], parts[1][0], parts[1][1],
                 W3[:NODE_DIM], W3[NODE_DIM:], b3.reshape(1, -1))


# speedup vs baseline: 1.0518x; 1.0518x over previous
import functools

import jax
import jax.numpy as jnp
from jax import lax
from jax.experimental import pallas as pl
from jax.experimental.pallas import tpu as pltpu
from jax.experimental.pallas import tpu_sc as plsc

NODE_DIM = 128
EDGE_DIM = 16
N_NODES = 10000
N_EDGES = 320000

NC = 2
NS = 16
NW = NC * NS
NSPLIT = 2
E_CHU = N_EDGES // NSPLIT
PER_W = E_CHU // NW
CH = 40
NCHUNK = PER_W // CH
ROWS_PER_SUB = 624
TAIL_ROWS = N_NODES - NS * ROWS_PER_SUB
TAIL_OFF = NS * ROWS_PER_SUB

BE = 4000
BN = 1000

def _mesh():
    return plsc.VectorSubcoreMesh(
        core_axis_name="c", subcore_axis_name="s", num_cores=NC, num_subcores=NS)


_NEG_LOG2E = -1.4426950408889634
_LN2 = 0.6931471805599453


def _softplus(v):
    e = jnp.exp2(jnp.abs(v) * _NEG_LOG2E)
    return jnp.maximum(v, 0.0) + jnp.log2(1.0 + e) * _LN2


DEPTH = 4


def _gather_body(x_hbm, row_hbm, col_hbm, xr_hbm, xc_hbm,
                 idx_r, idx_c,
                 br0, bc0, br1, bc1, br2, bc2, br3, bc3,
                 sr0, sc0, sr1, sc1, sr2, sc2, sr3, sc3):
    c = lax.axis_index("c")
    s = lax.axis_index("s")
    wid = s * NC + c
    base = wid * PER_W
    pltpu.sync_copy(row_hbm.at[wid], idx_r)
    pltpu.sync_copy(col_hbm.at[wid], idx_c)

    bufs = ((br0, bc0, sr0, sc0), (br1, bc1, sr1, sc1),
            (br2, bc2, sr2, sc2), (br3, bc3, sr3, sc3))

    def fire(j, k):
        br, bc, sr, sc = bufs[k]
        pltpu.async_copy(x_hbm.at[idx_r.at[j]], br, sr)
        pltpu.async_copy(x_hbm.at[idx_c.at[j]], bc, sc)

    def drain_write(j, k):
        br, bc, sr, sc = bufs[k]
        pltpu.make_async_copy(x_hbm.at[idx_r.at[j]], br, sr).wait()
        pltpu.make_async_copy(x_hbm.at[idx_c.at[j]], bc, sc).wait()
        off = base + j * CH
        pltpu.sync_copy(br, xr_hbm.at[pl.ds(off, CH)])
        pltpu.sync_copy(bc, xc_hbm.at[pl.ds(off, CH)])

    for k in range(DEPTH):
        fire(k, k)

    ngrp = NCHUNK // DEPTH

    def body(t, carry):
        for u in range(DEPTH):
            j = t * DEPTH + u
            drain_write(j, u)

            @pl.when(j + DEPTH < NCHUNK)
            def _():
                fire(j + DEPTH, u)
        return carry

    lax.fori_loop(0, ngrp, body, 0)
    for u in range(NCHUNK - ngrp * DEPTH):
        drain_write(ngrp * DEPTH + u, u)


@jax.jit
def _gather(x, row3, col3):
    f = pl.kernel(
        _gather_body,
        out_type=(
            jax.ShapeDtypeStruct((E_CHU, NODE_DIM), jnp.float32),
            jax.ShapeDtypeStruct((E_CHU, NODE_DIM), jnp.float32),
        ),
        mesh=_mesh(),
        scratch_types=(
            [pltpu.VMEM((NCHUNK, CH), jnp.int32)] * 2
            + [pltpu.VMEM((CH, NODE_DIM), jnp.float32)] * (2 * DEPTH)
            + [pltpu.SemaphoreType.DMA] * (2 * DEPTH)
        ),
    )
    return f(x, row3, col3)


def _scatter_body(emb_hbm, col_hbm, zeros_hbm, out_hbm,
                  idx_v, b0, b1, b2, b3, shared, s0, s1, s2, s3):
    c = lax.axis_index("c")
    s = lax.axis_index("s")
    wid = s * NC + c
    base = wid * PER_W
    r0 = s * ROWS_PER_SUB
    pltpu.sync_copy(zeros_hbm.at[pl.ds(r0, ROWS_PER_SUB)],
                    shared.at[pl.ds(r0, ROWS_PER_SUB)])

    @pl.when(s == NS - 1)
    def _():
        pltpu.sync_copy(zeros_hbm.at[pl.ds(TAIL_OFF, TAIL_ROWS)],
                        shared.at[pl.ds(TAIL_OFF, TAIL_ROWS)])

    pltpu.sync_copy(col_hbm.at[wid], idx_v)
    plsc.subcore_barrier()

    bufs = ((b0, s0), (b1, s1), (b2, s2), (b3, s3))

    def fire(j, k):
        b, sm = bufs[k]
        pltpu.async_copy(emb_hbm.at[pl.ds(base + j * CH, CH)], b, sm)

    def drain_add(j, k):
        b, sm = bufs[k]
        pltpu.make_async_copy(emb_hbm.at[pl.ds(base + j * CH, CH)], b,
                              sm).wait()
        pltpu.sync_copy(b, shared.at[idx_v.at[j]], add=True)

    for k in range(DEPTH):
        fire(k, k)

    ngrp = NCHUNK // DEPTH

    def body(t, carry):
        for u in range(DEPTH):
            j = t * DEPTH + u
            drain_add(j, u)

            @pl.when(j + DEPTH < NCHUNK)
            def _():
                fire(j + DEPTH, u)
        return carry

    lax.fori_loop(0, ngrp, body, 0)
    for u in range(NCHUNK - ngrp * DEPTH):
        drain_add(ngrp * DEPTH + u, u)
    plsc.subcore_barrier()
    pltpu.sync_copy(shared.at[pl.ds(r0, ROWS_PER_SUB)],
                    out_hbm.at[c, pl.ds(r0, ROWS_PER_SUB)])

    @pl.when(s == NS - 1)
    def _():
        pltpu.sync_copy(shared.at[pl.ds(TAIL_OFF, TAIL_ROWS)],
                        out_hbm.at[c, pl.ds(TAIL_OFF, TAIL_ROWS)])


@jax.jit
def _scatter(emb, col3, zeros):
    f = pl.kernel(
        _scatter_body,
        out_type=jax.ShapeDtypeStruct((NC, N_NODES, NODE_DIM), jnp.float32),
        mesh=_mesh(),
        scratch_types=(
            [pltpu.VMEM((NCHUNK, CH), jnp.int32)]
            + [pltpu.VMEM((CH, NODE_DIM), jnp.float32)] * DEPTH
            + [pltpu.VMEM_SHARED((N_NODES, NODE_DIM), jnp.float32)]
            + [pltpu.SemaphoreType.DMA] * DEPTH
        ),
    )
    return f(emb, col3, zeros)


def _edge_mlp_body(xr, xc, ea, w1a, w1b, w1c, b1, w2, b2, out):
    bf = jnp.bfloat16
    acc = jnp.dot(xr[...].astype(bf), w1a[...],
                  preferred_element_type=jnp.float32)
    acc += jnp.dot(xc[...].astype(bf), w1b[...],
                   preferred_element_type=jnp.float32)
    acc += jnp.dot(ea[...], w1c[...], preferred_element_type=jnp.float32)
    acc += b1[...]
    h = _softplus(acc)
    o = jnp.dot(h.astype(jnp.bfloat16), w2[...],
                preferred_element_type=jnp.float32) + b2[...]
    out[...] = _softplus(o)


@jax.jit
def _edge_mlp(xr, xc, ea, w1a, w1b, w1c, b1, w2, b2):
    nblk = E_CHU // BE
    full = lambda shape: pl.BlockSpec(shape, lambda i: (0, 0))
    return pl.pallas_call(
        _edge_mlp_body,
        grid=(nblk,),
        in_specs=[
            pl.BlockSpec((BE, NODE_DIM), lambda i: (i, 0)),
            pl.BlockSpec((BE, NODE_DIM), lambda i: (i, 0)),
            pl.BlockSpec((BE, EDGE_DIM), lambda i: (i, 0)),
            full((NODE_DIM, 2 * NODE_DIM)),
            full((NODE_DIM, 2 * NODE_DIM)),
            full((EDGE_DIM, 2 * NODE_DIM)),
            full((1, 2 * NODE_DIM)),
            full((2 * NODE_DIM, NODE_DIM)),
            full((1, NODE_DIM)),
        ],
        out_specs=pl.BlockSpec((BE, NODE_DIM), lambda i: (i, 0)),
        out_shape=jax.ShapeDtypeStruct((E_CHU, NODE_DIM), jnp.float32),
        compiler_params=pltpu.CompilerParams(
            dimension_semantics=("parallel",)),
    )(xr, xc, ea, w1a, w1b, w1c, b1, w2, b2)


def _node_body(x, a0, a1, a2, a3, w3a, w3b, b3, out):
    ag = (a0[...] + a1[...]) + (a2[...] + a3[...])
    o = jnp.dot(x[...], w3a[...], preferred_element_type=jnp.float32)
    o += jnp.dot(ag, w3b[...], preferred_element_type=jnp.float32)
    o += b3[...]
    out[...] = _softplus(o) + x[...]


@jax.jit
def _node(x, a0, a1, a2, a3, w3a, w3b, b3):
    nblk = N_NODES // BN
    full = lambda shape: pl.BlockSpec(shape, lambda i: (0, 0))
    blk = pl.BlockSpec((BN, NODE_DIM), lambda i: (i, 0))
    return pl.pallas_call(
        _node_body,
        grid=(nblk,),
        in_specs=[
            blk, blk, blk, blk, blk,
            full((NODE_DIM, NODE_DIM)),
            full((NODE_DIM, NODE_DIM)),
            full((1, NODE_DIM)),
        ],
        out_specs=pl.BlockSpec((BN, NODE_DIM), lambda i: (i, 0)),
        out_shape=jax.ShapeDtypeStruct((N_NODES, NODE_DIM), jnp.float32),
        compiler_params=pltpu.CompilerParams(
            dimension_semantics=("parallel",)),
    )(x, a0, a1, a2, a3, w3a, w3b, b3)


def kernel(x, edge_index, edge_attr, W1, b1, W2, b2, W3, b3):
    row4 = edge_index[0].astype(jnp.int32).reshape(NSPLIT, NW, NCHUNK, CH)
    col4 = edge_index[1].astype(jnp.int32).reshape(NSPLIT, NW, NCHUNK, CH)
    bf = jnp.bfloat16
    w1a = W1[:NODE_DIM].astype(bf)
    w1b = W1[NODE_DIM:2 * NODE_DIM].astype(bf)
    w1c = W1[2 * NODE_DIM:].astype(bf)
    b1r = b1.reshape(1, -1)
    w2 = W2.astype(bf)
    b2r = b2.reshape(1, -1)
    ea = edge_attr.astype(bf)
    zeros = jnp.zeros((N_NODES, NODE_DIM), jnp.float32)

    parts = []
    for t in range(NSPLIT):
        xr, xc = _gather(x, row4[t], col4[t])
        emb = _edge_mlp(xr, xc, ea[t * E_CHU:(t + 1) * E_CHU],
                        w1a, w1b, w1c, b1r, w2, b2r)
        parts.append(_scatter(emb, col4[t], zeros))
    return _node(x, parts[0][0], parts[0][1], parts[1][0], parts[1][1],
                 W3[:NODE_DIM], W3[NODE_DIM:], b3.reshape(1, -1))
